# baseline (speedup 1.0000x reference)
from functools import partial

import jax
import jax.numpy as jnp
from jax.experimental import pallas as pl
from jax.experimental.pallas import tpu as pltpu

_EPS = 1e-5


def _sigmoid(x):
    return pl.reciprocal(1.0 + jnp.exp(-x))


def _zero_border(pad_ref, H2, W2, C, dtype):
    pad_ref[0:1, :, :] = jnp.zeros((1, W2 + 2, C), dtype)
    pad_ref[H2 + 1:H2 + 2, :, :] = jnp.zeros((1, W2 + 2, C), dtype)
    pad_ref[:, 0:1, :] = jnp.zeros((H2 + 2, 1, C), dtype)
    pad_ref[:, W2 + 1:W2 + 2, :] = jnp.zeros((H2 + 2, 1, C), dtype)


def _taps(pad_ref, H2, W2, C):
    taps = []
    for ki in range(3):
        for kj in range(3):
            taps.append(pad_ref[ki:ki + H2, kj:kj + W2, :])
    return jnp.concatenate(taps, axis=-1).reshape(H2 * W2, 9 * C)


def _pool_conv1_kernel(x_ref, w_ref, y_ref, st_ref, pad_ref,
                       *, H2, W2, CIN, COUT):
    x = x_ref[0]
    pooled = jnp.max(jnp.max(x, axis=3), axis=1)

    _zero_border(pad_ref, H2, W2, CIN, jnp.bfloat16)
    pad_ref[1:H2 + 1, 1:W2 + 1, :] = pooled.astype(jnp.bfloat16)

    lhs = _taps(pad_ref, H2, W2, CIN)
    acc = jnp.dot(lhs, w_ref[...], preferred_element_type=jnp.float32)
    y_ref[0] = acc.astype(jnp.bfloat16)
    st_ref[0] = jnp.concatenate(
        [jnp.sum(acc, axis=0, keepdims=True),
         jnp.sum(acc * acc, axis=0, keepdims=True)], axis=0)


def _fold_rowstats(st, g, b, inv_cnt):
    mean = st[0:1, :] * inv_cnt
    var = st[1:2, :] * inv_cnt - mean * mean
    scale = g * jax.lax.rsqrt(var + _EPS)
    shift = b - mean * scale
    return scale, shift


def _bn_sig_conv2_kernel(y1_ref, st1_ref, g_ref, b_ref, w_ref, y_ref, st_ref,
                         pad_ref, *, H2, W2, COUT, inv_cnt):
    scale, shift = _fold_rowstats(jnp.sum(st1_ref[...], axis=0),
                                  g_ref[...], b_ref[...], inv_cnt)
    h = _sigmoid(y1_ref[0].astype(jnp.float32) * scale + shift)

    _zero_border(pad_ref, H2, W2, COUT, jnp.bfloat16)
    pad_ref[1:H2 + 1, 1:W2 + 1, :] = h.astype(jnp.bfloat16).reshape(H2, W2, COUT)

    rhs = _taps(pad_ref, H2, W2, COUT)
    acc = jax.lax.dot_general(
        w_ref[...], rhs, (((0,), (1,)), ((), ())),
        preferred_element_type=jnp.float32)
    y_ref[0] = acc.astype(jnp.bfloat16)
    st_ref[0] = jnp.concatenate(
        [jnp.sum(acc, axis=1, keepdims=True),
         jnp.sum(acc * acc, axis=1, keepdims=True)], axis=1)


def _bn_sig_out_kernel(y2_ref, st2_ref, g_ref, b_ref, out_ref, *, inv_cnt):
    st = jnp.sum(st2_ref[...], axis=0)
    mean = st[:, 0:1] * inv_cnt
    var = st[:, 1:2] * inv_cnt - mean * mean
    scale = g_ref[...] * jax.lax.rsqrt(var + _EPS)
    shift = b_ref[...] - mean * scale
    out_ref[0] = _sigmoid(y2_ref[0].astype(jnp.float32) * scale + shift)


def kernel(x_nchw, w1_hwio, g1, b1, w2_hwio, g2, b2):
    N, CIN, H, W = x_nchw.shape
    COUT = w1_hwio.shape[-1]
    H2, W2 = H // 2, W // 2
    M = H2 * W2
    inv_cnt = 1.0 / float(N * M)

    x_nhwc = jnp.transpose(x_nchw, (0, 2, 3, 1)).astype(jnp.float32)
    x6 = x_nhwc.reshape(N, H2, 2, W2, 2, CIN)
    w1k = w1_hwio.reshape(9 * CIN, COUT).astype(jnp.bfloat16)
    w2k = w2_hwio.reshape(9 * COUT, COUT).astype(jnp.bfloat16)
    g1r = g1.reshape(1, COUT).astype(jnp.float32)
    b1r = b1.reshape(1, COUT).astype(jnp.float32)
    g2c = g2.reshape(COUT, 1).astype(jnp.float32)
    b2c = b2.reshape(COUT, 1).astype(jnp.float32)

    cparams = pltpu.CompilerParams(
        dimension_semantics=("parallel",),
        vmem_limit_bytes=48 * 1024 * 1024,
    )

    y1_shape = jax.ShapeDtypeStruct((N, M, COUT), jnp.bfloat16)
    st1_shape = jax.ShapeDtypeStruct((N, 2, COUT), jnp.float32)
    y2_shape = jax.ShapeDtypeStruct((N, COUT, M), jnp.bfloat16)
    st2_shape = jax.ShapeDtypeStruct((N, COUT, 2), jnp.float32)
    out_shape = jax.ShapeDtypeStruct((N, COUT, M), jnp.float32)

    y1_spec = pl.BlockSpec((1, M, COUT), lambda i: (i, 0, 0))
    st1_spec = pl.BlockSpec((1, 2, COUT), lambda i: (i, 0, 0))
    y2_spec = pl.BlockSpec((1, COUT, M), lambda i: (i, 0, 0))
    st2_spec = pl.BlockSpec((1, COUT, 2), lambda i: (i, 0, 0))

    y1, st1 = pl.pallas_call(
        partial(_pool_conv1_kernel, H2=H2, W2=W2, CIN=CIN, COUT=COUT),
        grid=(N,),
        in_specs=[
            pl.BlockSpec((1, H2, 2, W2, 2, CIN), lambda i: (i, 0, 0, 0, 0, 0)),
            pl.BlockSpec((9 * CIN, COUT), lambda i: (0, 0)),
        ],
        out_specs=[y1_spec, st1_spec],
        out_shape=(y1_shape, st1_shape),
        scratch_shapes=[pltpu.VMEM((H2 + 2, W2 + 2, CIN), jnp.bfloat16)],
        compiler_params=cparams,
    )(x6, w1k)

    y2, st2 = pl.pallas_call(
        partial(_bn_sig_conv2_kernel, H2=H2, W2=W2, COUT=COUT, inv_cnt=inv_cnt),
        grid=(N,),
        in_specs=[
            y1_spec,
            pl.BlockSpec((N, 2, COUT), lambda i: (0, 0, 0)),
            pl.BlockSpec((1, COUT), lambda i: (0, 0)),
            pl.BlockSpec((1, COUT), lambda i: (0, 0)),
            pl.BlockSpec((9 * COUT, COUT), lambda i: (0, 0)),
        ],
        out_specs=[y2_spec, st2_spec],
        out_shape=(y2_shape, st2_shape),
        scratch_shapes=[pltpu.VMEM((H2 + 2, W2 + 2, COUT), jnp.bfloat16)],
        compiler_params=cparams,
    )(y1, st1, g1r, b1r, w2k)

    out_flat = pl.pallas_call(
        partial(_bn_sig_out_kernel, inv_cnt=inv_cnt),
        grid=(N,),
        in_specs=[
            y2_spec,
            pl.BlockSpec((N, COUT, 2), lambda i: (0, 0, 0)),
            pl.BlockSpec((COUT, 1), lambda i: (0, 0)),
            pl.BlockSpec((COUT, 1), lambda i: (0, 0)),
        ],
        out_specs=pl.BlockSpec((1, COUT, M), lambda i: (i, 0, 0)),
        out_shape=out_shape,
        compiler_params=cparams,
    )(y2, st2, g2c, b2c)

    return out_flat.reshape(N, COUT, H2, W2)


# speedup vs baseline: 1.3934x; 1.3934x over previous
from functools import partial

import jax
import jax.numpy as jnp
from jax.experimental import pallas as pl
from jax.experimental.pallas import tpu as pltpu

_EPS = 1e-5


def _sigmoid(x):
    return pl.reciprocal(1.0 + jnp.exp(-x))


def _zero_border(pad_ref, H2, W2, C, dtype):
    pad_ref[0:1, :, :] = jnp.zeros((1, W2 + 2, C), dtype)
    pad_ref[H2 + 1:H2 + 2, :, :] = jnp.zeros((1, W2 + 2, C), dtype)
    pad_ref[:, 0:1, :] = jnp.zeros((H2 + 2, 1, C), dtype)
    pad_ref[:, W2 + 1:W2 + 2, :] = jnp.zeros((H2 + 2, 1, C), dtype)


def _taps(pad_ref, H2, W2, C):
    taps = []
    for ki in range(3):
        for kj in range(3):
            taps.append(pad_ref[ki:ki + H2, kj:kj + W2, :])
    return jnp.concatenate(taps, axis=-1).reshape(H2 * W2, 9 * C)


def _pool_conv1_kernel(x_ref, eye_ref, w_ref, y_ref, st_ref, pad_ref,
                       *, H2, W2, CIN, COUT):
    x = x_ref[0]
    W = 2 * W2
    hm = jnp.maximum(x[:, :, :W], x[:, :, W:])
    hmb = hm.astype(jnp.bfloat16).reshape(CIN, H2 * W)
    t = jax.lax.dot_general(hmb, eye_ref[...], (((0,), (0,)), ((), ())),
                            preferred_element_type=jnp.float32)
    tp = t.reshape(H2 * W2, 2, CIN)
    pooled = jnp.maximum(tp[:, 0, :], tp[:, 1, :]).astype(jnp.bfloat16)

    _zero_border(pad_ref, H2, W2, CIN, jnp.bfloat16)
    pad_ref[1:H2 + 1, 1:W2 + 1, :] = pooled.reshape(H2, W2, CIN)

    lhs = _taps(pad_ref, H2, W2, CIN)
    acc = jnp.dot(lhs, w_ref[...], preferred_element_type=jnp.float32)
    y_ref[0] = acc.astype(jnp.bfloat16)
    st_ref[0] = jnp.concatenate(
        [jnp.sum(acc, axis=0, keepdims=True),
         jnp.sum(acc * acc, axis=0, keepdims=True)], axis=0)


def _fold_rowstats(st, g, b, inv_cnt):
    mean = st[0:1, :] * inv_cnt
    var = st[1:2, :] * inv_cnt - mean * mean
    scale = g * jax.lax.rsqrt(var + _EPS)
    shift = b - mean * scale
    return scale, shift


def _bn_sig_conv2_kernel(y1_ref, st1_ref, g_ref, b_ref, w_ref, y_ref, st_ref,
                         pad_ref, *, H2, W2, COUT, inv_cnt):
    scale, shift = _fold_rowstats(jnp.sum(st1_ref[...], axis=0),
                                  g_ref[...], b_ref[...], inv_cnt)
    h = _sigmoid(y1_ref[0].astype(jnp.float32) * scale + shift)

    _zero_border(pad_ref, H2, W2, COUT, jnp.bfloat16)
    pad_ref[1:H2 + 1, 1:W2 + 1, :] = h.astype(jnp.bfloat16).reshape(H2, W2, COUT)

    rhs = _taps(pad_ref, H2, W2, COUT)
    acc = jax.lax.dot_general(
        w_ref[...], rhs, (((0,), (1,)), ((), ())),
        preferred_element_type=jnp.float32)
    y_ref[0] = acc.astype(jnp.bfloat16)
    st_ref[0] = jnp.concatenate(
        [jnp.sum(acc, axis=1, keepdims=True),
         jnp.sum(acc * acc, axis=1, keepdims=True)], axis=1)


def _bn_sig_out_kernel(y2_ref, st2_ref, g_ref, b_ref, out_ref, *, inv_cnt):
    st = jnp.sum(st2_ref[...], axis=0)
    mean = st[:, 0:1] * inv_cnt
    var = st[:, 1:2] * inv_cnt - mean * mean
    scale = g_ref[...] * jax.lax.rsqrt(var + _EPS)
    shift = b_ref[...] - mean * scale
    out_ref[0] = _sigmoid(y2_ref[0].astype(jnp.float32) * scale + shift)


def kernel(x_nchw, w1_hwio, g1, b1, w2_hwio, g2, b2):
    N, CIN, H, W = x_nchw.shape
    COUT = w1_hwio.shape[-1]
    H2, W2 = H // 2, W // 2
    M = H2 * W2
    inv_cnt = 1.0 / float(N * M)

    xv = x_nchw.reshape(N, CIN, H2, 2 * W)
    eye = jnp.eye(CIN, dtype=jnp.bfloat16)
    w1k = w1_hwio.reshape(9 * CIN, COUT).astype(jnp.bfloat16)
    w2k = w2_hwio.reshape(9 * COUT, COUT).astype(jnp.bfloat16)
    g1r = g1.reshape(1, COUT).astype(jnp.float32)
    b1r = b1.reshape(1, COUT).astype(jnp.float32)
    g2c = g2.reshape(COUT, 1).astype(jnp.float32)
    b2c = b2.reshape(COUT, 1).astype(jnp.float32)

    cparams = pltpu.CompilerParams(
        dimension_semantics=("parallel",),
        vmem_limit_bytes=48 * 1024 * 1024,
    )

    y1_shape = jax.ShapeDtypeStruct((N, M, COUT), jnp.bfloat16)
    st1_shape = jax.ShapeDtypeStruct((N, 2, COUT), jnp.float32)
    y2_shape = jax.ShapeDtypeStruct((N, COUT, M), jnp.bfloat16)
    st2_shape = jax.ShapeDtypeStruct((N, COUT, 2), jnp.float32)
    out_shape = jax.ShapeDtypeStruct((N, COUT, M), jnp.float32)

    y1_spec = pl.BlockSpec((1, M, COUT), lambda i: (i, 0, 0))
    st1_spec = pl.BlockSpec((1, 2, COUT), lambda i: (i, 0, 0))
    y2_spec = pl.BlockSpec((1, COUT, M), lambda i: (i, 0, 0))
    st2_spec = pl.BlockSpec((1, COUT, 2), lambda i: (i, 0, 0))

    y1, st1 = pl.pallas_call(
        partial(_pool_conv1_kernel, H2=H2, W2=W2, CIN=CIN, COUT=COUT),
        grid=(N,),
        in_specs=[
            pl.BlockSpec((1, CIN, H2, 2 * W), lambda i: (i, 0, 0, 0)),
            pl.BlockSpec((CIN, CIN), lambda i: (0, 0)),
            pl.BlockSpec((9 * CIN, COUT), lambda i: (0, 0)),
        ],
        out_specs=[y1_spec, st1_spec],
        out_shape=(y1_shape, st1_shape),
        scratch_shapes=[pltpu.VMEM((H2 + 2, W2 + 2, CIN), jnp.bfloat16)],
        compiler_params=cparams,
    )(xv, eye, w1k)

    y2, st2 = pl.pallas_call(
        partial(_bn_sig_conv2_kernel, H2=H2, W2=W2, COUT=COUT, inv_cnt=inv_cnt),
        grid=(N,),
        in_specs=[
            y1_spec,
            pl.BlockSpec((N, 2, COUT), lambda i: (0, 0, 0)),
            pl.BlockSpec((1, COUT), lambda i: (0, 0)),
            pl.BlockSpec((1, COUT), lambda i: (0, 0)),
            pl.BlockSpec((9 * COUT, COUT), lambda i: (0, 0)),
        ],
        out_specs=[y2_spec, st2_spec],
        out_shape=(y2_shape, st2_shape),
        scratch_shapes=[pltpu.VMEM((H2 + 2, W2 + 2, COUT), jnp.bfloat16)],
        compiler_params=cparams,
    )(y1, st1, g1r, b1r, w2k)

    out_flat = pl.pallas_call(
        partial(_bn_sig_out_kernel, inv_cnt=inv_cnt),
        grid=(N,),
        in_specs=[
            y2_spec,
            pl.BlockSpec((N, COUT, 2), lambda i: (0, 0, 0)),
            pl.BlockSpec((COUT, 1), lambda i: (0, 0)),
            pl.BlockSpec((COUT, 1), lambda i: (0, 0)),
        ],
        out_specs=pl.BlockSpec((1, COUT, M), lambda i: (i, 0, 0)),
        out_shape=out_shape,
        compiler_params=cparams,
    )(y2, st2, g2c, b2c)

    return out_flat.reshape(N, COUT, H2, W2)
